# baseline (speedup 1.0000x reference)
import functools

import jax
import jax.numpy as jnp
from jax import lax
from jax.experimental import pallas as pl
from jax.experimental.pallas import tpu as pltpu
from jax.experimental.pallas import tpu_sc as plsc

B, F, TOK = 4096, 26, 20
DIM = 32
LANES = 16
NC, NS = 2, 16
NW = NC * NS
B_PW = B // NW
GB = 64
C = GB * TOK
NCHUNK = F * (B_PW // GB)
EPS = 1e-12
RSQRT_MAGIC = 0x5F3759DF

_GDN = lax.GatherDimensionNumbers(
    offset_dims=(), collapsed_slice_dims=(0,), start_index_map=(0,))


def _lane_perm(v, idx):
    return lax.gather(v, idx.reshape(LANES, 1), _GDN, (1,),
                      mode=lax.GatherScatterMode.PROMISE_IN_BOUNDS)


def _newton_rsqrt(x):
    i = plsc.bitcast(x, jnp.int32)
    i = RSQRT_MAGIC - lax.shift_right_logical(i, 1)
    y = plsc.bitcast(i, jnp.float32)
    y = y * (1.5 - (x * 0.5) * y * y)
    return y


def _make_sc_kernel():
    mesh = plsc.VectorSubcoreMesh(core_axis_name="c", subcore_axis_name="s")

    @functools.partial(
        pl.kernel,
        out_type=jax.ShapeDtypeStruct((B, F, DIM), jnp.float32),
        mesh=mesh,
        compiler_params=pltpu.CompilerParams(
            needs_layout_passes=False, use_tc_tiling_on_sc=False),
        scratch_types=[
            pltpu.VMEM((TOK, GB), jnp.int32),
            pltpu.VMEM((TOK, GB), jnp.int32),
            pltpu.VMEM((C, DIM), jnp.float32),
            pltpu.VMEM((C, DIM), jnp.float32),
            pltpu.VMEM((GB, 1, DIM), jnp.float32),
            pltpu.VMEM((GB, 1, DIM), jnp.float32),
            pltpu.VMEM((DIM,), jnp.float32),
            pltpu.VMEM((DIM,), jnp.float32),
            pltpu.SemaphoreType.DMA,
            pltpu.SemaphoreType.DMA,
            pltpu.SemaphoreType.DMA,
            pltpu.SemaphoreType.DMA,
        ],
    )
    def sc_kernel(ids_hbm, table_hbm, gamma_hbm, beta_hbm, out_hbm,
                  idx0, idx1, rows0, rows1, out0, out1, gam_v, bet_v,
                  isem, rsem, osem0, osem1):
        wid = lax.axis_index("s") * NC + lax.axis_index("c")
        base_b = wid * B_PW
        pltpu.sync_copy(gamma_hbm, gam_v)
        pltpu.sync_copy(beta_hbm, bet_v)
        glo = gam_v[pl.ds(0, LANES)]
        ghi = gam_v[pl.ds(LANES, LANES)]
        blo = bet_v[pl.ds(0, LANES)] * float(TOK)
        bhi = bet_v[pl.ds(LANES, LANES)] * float(TOK)
        idx15 = jnp.full((LANES,), LANES - 1, jnp.int32)

        idxs = (idx0, idx1)
        rows = (rows0, rows1)
        outs = (out0, out1)
        osems = (osem0, osem1)

        def split(c):
            return c // 2, base_b + (c % 2) * GB

        def idx_copies(c, b):
            f, b0 = split(c)
            return [
                pltpu.make_async_copy(
                    ids_hbm.at[f, l, pl.ds(b0, GB)],
                    idxs[b].at[l], isem)
                for l in range(TOK)
            ]

        def row_copies(c, b):
            return [
                pltpu.make_async_copy(
                    table_hbm.at[idxs[b].at[l]],
                    rows[b].at[pl.ds(l * GB, GB)],
                    rsem,
                )
                for l in range(TOK)
            ]

        def out_copy(c, b):
            f, b0 = split(c)
            return pltpu.make_async_copy(
                outs[b],
                out_hbm.at[pl.ds(b0, GB), pl.ds(f, 1)],
                osems[b])

        def lane_total(v):
            return _lane_perm(jnp.cumsum(v), idx15)

        def compute(b):
            rv = rows[b]
            ov = outs[b]

            def group(g, gcarry):
                acc_lo = jnp.zeros((LANES,), jnp.float32)
                acc_hi = jnp.zeros((LANES,), jnp.float32)
                for l in range(TOK):
                    r = l * GB + g
                    lo = rv[r, pl.ds(0, LANES)]
                    hi = rv[r, pl.ds(LANES, LANES)]
                    tot = lane_total(lo + hi)
                    tot2 = lane_total(lo * lo + hi * hi)
                    mean = tot * (1.0 / DIM)
                    var = tot2 * (1.0 / DIM) - mean * mean
                    inv = _newton_rsqrt(var + EPS)
                    acc_lo = acc_lo + (lo - mean) * inv
                    acc_hi = acc_hi + (hi - mean) * inv
                ov[g, 0, pl.ds(0, LANES)] = acc_lo * glo + blo
                ov[g, 0, pl.ds(LANES, LANES)] = acc_hi * ghi + bhi
                return gcarry

            lax.fori_loop(0, GB, group, 0)

        def pair(c2, carry):
            for b in (0, 1):
                c = c2 * 2 + b
                for cp in row_copies(c, b):
                    cp.wait()

                @pl.when(c + 1 < NCHUNK)
                def _():
                    for cp in idx_copies(c + 1, b ^ 1):
                        cp.wait()
                    for cp in row_copies(c + 1, b ^ 1):
                        cp.start()

                @pl.when(c >= 2)
                def _():
                    out_copy(c - 2, b).wait()

                compute(b)

                @pl.when(c + 2 < NCHUNK)
                def _():
                    for cp in idx_copies(c + 2, b):
                        cp.start()

                out_copy(c, b).start()
            return carry

        for cp in idx_copies(0, 0):
            cp.start()
        for cp in idx_copies(0, 0):
            cp.wait()
        for cp in row_copies(0, 0):
            cp.start()
        for cp in idx_copies(1, 1):
            cp.start()
        lax.fori_loop(0, NCHUNK // 2, pair, 0)
        out_copy(NCHUNK - 2, 0).wait()
        out_copy(NCHUNK - 1, 1).wait()

    return sc_kernel


_SC_KERNEL = _make_sc_kernel()


def kernel(input_ids, table, gamma, beta):
    ids_t = input_ids.transpose(1, 2, 0)
    return _SC_KERNEL(ids_t, table, gamma, beta)


# speedup vs baseline: 1.0234x; 1.0234x over previous
import functools

import jax
import jax.numpy as jnp
from jax import lax
from jax.experimental import pallas as pl
from jax.experimental.pallas import tpu as pltpu
from jax.experimental.pallas import tpu_sc as plsc

B, F, TOK = 4096, 26, 20
DIM = 32
LANES = 16
NC, NS = 2, 16
NW = NC * NS
B_PW = B // NW
GB = 64
C = GB * TOK
NCHUNK = F * (B_PW // GB)
EPS = 1e-12
RSQRT_MAGIC = 0x5F3759DF

_GDN = lax.GatherDimensionNumbers(
    offset_dims=(), collapsed_slice_dims=(0,), start_index_map=(0,))


def _lane_perm(v, idx):
    return lax.gather(v, idx.reshape(LANES, 1), _GDN, (1,),
                      mode=lax.GatherScatterMode.PROMISE_IN_BOUNDS)


def _newton_rsqrt_half(xh):
    i = plsc.bitcast(xh, jnp.int32)
    i = (RSQRT_MAGIC - 0x400000) - lax.shift_right_logical(i, 1)
    y = plsc.bitcast(i, jnp.float32)
    y = y * (1.5 - xh * y * y)
    return y


def _make_sc_kernel():
    mesh = plsc.VectorSubcoreMesh(core_axis_name="c", subcore_axis_name="s")

    @functools.partial(
        pl.kernel,
        out_type=jax.ShapeDtypeStruct((B, F, DIM), jnp.float32),
        mesh=mesh,
        compiler_params=pltpu.CompilerParams(
            needs_layout_passes=False, use_tc_tiling_on_sc=False),
        scratch_types=[
            pltpu.VMEM((TOK, GB), jnp.int32),
            pltpu.VMEM((TOK, GB), jnp.int32),
            pltpu.VMEM((C, DIM), jnp.float32),
            pltpu.VMEM((C, DIM), jnp.float32),
            pltpu.VMEM((GB, 1, DIM), jnp.float32),
            pltpu.VMEM((GB, 1, DIM), jnp.float32),
            pltpu.VMEM((DIM,), jnp.float32),
            pltpu.VMEM((DIM,), jnp.float32),
            pltpu.SemaphoreType.DMA,
            pltpu.SemaphoreType.DMA,
            pltpu.SemaphoreType.DMA,
            pltpu.SemaphoreType.DMA,
        ],
    )
    def sc_kernel(ids_hbm, table_hbm, gamma_hbm, beta_hbm, out_hbm,
                  idx0, idx1, rows0, rows1, out0, out1, gam_v, bet_v,
                  isem, rsem, osem0, osem1):
        wid = lax.axis_index("s") * NC + lax.axis_index("c")
        base_b = wid * B_PW
        pltpu.sync_copy(gamma_hbm, gam_v)
        pltpu.sync_copy(beta_hbm, bet_v)
        glo = gam_v[pl.ds(0, LANES)]
        ghi = gam_v[pl.ds(LANES, LANES)]
        blo = bet_v[pl.ds(0, LANES)] * float(TOK)
        bhi = bet_v[pl.ds(LANES, LANES)] * float(TOK)
        idx15 = jnp.full((LANES,), LANES - 1, jnp.int32)

        idxs = (idx0, idx1)
        rows = (rows0, rows1)
        outs = (out0, out1)
        osems = (osem0, osem1)

        def split(c):
            return c // 2, base_b + (c % 2) * GB

        def idx_copies(c, b):
            f, b0 = split(c)
            return [
                pltpu.make_async_copy(
                    ids_hbm.at[f, l, pl.ds(b0, GB)],
                    idxs[b].at[l], isem)
                for l in range(TOK)
            ]

        def row_copies(c, b):
            return [
                pltpu.make_async_copy(
                    table_hbm.at[idxs[b].at[l]],
                    rows[b].at[pl.ds(l * GB, GB)],
                    rsem,
                )
                for l in range(TOK)
            ]

        def out_copy(c, b):
            f, b0 = split(c)
            return pltpu.make_async_copy(
                outs[b],
                out_hbm.at[pl.ds(b0, GB), pl.ds(f, 1)],
                osems[b])

        def lane_total(v):
            return _lane_perm(jnp.cumsum(v), idx15)

        def compute(b):
            rv = rows[b]
            ov = outs[b]

            def group(g, gcarry):
                acc_lo = jnp.zeros((LANES,), jnp.float32)
                acc_hi = jnp.zeros((LANES,), jnp.float32)
                for l in range(TOK):
                    r = l * GB + g
                    lo = rv[r, pl.ds(0, LANES)]
                    hi = rv[r, pl.ds(LANES, LANES)]
                    tot = lane_total(lo + hi)
                    tot2 = lane_total(lo * lo + hi * hi)
                    mean = tot * (1.0 / DIM)
                    mh = tot * (0.5 / DIM)
                    xh = tot2 * (0.5 / DIM) - mean * mh
                    inv = _newton_rsqrt_half(xh)
                    acc_lo = acc_lo + (lo - mean) * inv
                    acc_hi = acc_hi + (hi - mean) * inv
                ov[g, 0, pl.ds(0, LANES)] = acc_lo * glo + blo
                ov[g, 0, pl.ds(LANES, LANES)] = acc_hi * ghi + bhi
                return gcarry

            lax.fori_loop(0, GB, group, 0)

        def pair(c2, carry):
            for b in (0, 1):
                c = c2 * 2 + b
                for cp in row_copies(c, b):
                    cp.wait()

                @pl.when(c + 1 < NCHUNK)
                def _():
                    for cp in idx_copies(c + 1, b ^ 1):
                        cp.wait()
                    for cp in row_copies(c + 1, b ^ 1):
                        cp.start()

                @pl.when(c >= 2)
                def _():
                    out_copy(c - 2, b).wait()

                compute(b)

                @pl.when(c + 2 < NCHUNK)
                def _():
                    for cp in idx_copies(c + 2, b):
                        cp.start()

                out_copy(c, b).start()
            return carry

        for cp in idx_copies(0, 0):
            cp.start()
        for cp in idx_copies(0, 0):
            cp.wait()
        for cp in row_copies(0, 0):
            cp.start()
        for cp in idx_copies(1, 1):
            cp.start()
        lax.fori_loop(0, NCHUNK // 2, pair, 0)
        out_copy(NCHUNK - 2, 0).wait()
        out_copy(NCHUNK - 1, 1).wait()

    return sc_kernel


_SC_KERNEL = _make_sc_kernel()


def kernel(input_ids, table, gamma, beta):
    ids_t = input_ids.transpose(1, 2, 0)
    return _SC_KERNEL(ids_t, table, gamma, beta)


# speedup vs baseline: 1.0444x; 1.0205x over previous
import functools

import jax
import jax.numpy as jnp
from jax import lax
from jax.experimental import pallas as pl
from jax.experimental.pallas import tpu as pltpu
from jax.experimental.pallas import tpu_sc as plsc

B, F, TOK = 4096, 26, 20
DIM = 32
LANES = 16
NC, NS = 2, 16
NW = NC * NS
B_PW = B // NW
GB = 64
C = GB * TOK
NCHUNK = F * (B_PW // GB)
EPS = 1e-12
RSQRT_MAGIC = 0x5F3759DF

_GDN = lax.GatherDimensionNumbers(
    offset_dims=(), collapsed_slice_dims=(0,), start_index_map=(0,))


def _lane_perm(v, idx):
    return lax.gather(v, idx.reshape(LANES, 1), _GDN, (1,),
                      mode=lax.GatherScatterMode.PROMISE_IN_BOUNDS)


def _newton_rsqrt_half(xh):
    i = plsc.bitcast(xh, jnp.int32)
    i = (RSQRT_MAGIC - 0x400000) - lax.shift_right_logical(i, 1)
    y = plsc.bitcast(i, jnp.float32)
    y = y * (1.5 - xh * y * y)
    return y


def _make_sc_kernel():
    mesh = plsc.VectorSubcoreMesh(core_axis_name="c", subcore_axis_name="s")

    @functools.partial(
        pl.kernel,
        out_type=jax.ShapeDtypeStruct((B, F, DIM), jnp.float32),
        mesh=mesh,
        compiler_params=pltpu.CompilerParams(
            needs_layout_passes=False, use_tc_tiling_on_sc=False),
        scratch_types=[
            pltpu.VMEM((TOK, GB), jnp.int32),
            pltpu.VMEM((TOK, GB), jnp.int32),
            pltpu.VMEM((C, DIM), jnp.float32),
            pltpu.VMEM((C, DIM), jnp.float32),
            pltpu.VMEM((GB, 1, DIM), jnp.float32),
            pltpu.VMEM((GB, 1, DIM), jnp.float32),
            pltpu.VMEM((DIM,), jnp.float32),
            pltpu.VMEM((DIM,), jnp.float32),
            pltpu.SemaphoreType.DMA,
            pltpu.SemaphoreType.DMA,
            pltpu.SemaphoreType.DMA,
            pltpu.SemaphoreType.DMA,
        ],
    )
    def sc_kernel(ids_hbm, table_hbm, gamma_hbm, beta_hbm, out_hbm,
                  idx0, idx1, rows0, rows1, out0, out1, gam_v, bet_v,
                  isem, rsem, osem0, osem1):
        wid = lax.axis_index("s") * NC + lax.axis_index("c")
        base_b = wid * B_PW
        pltpu.sync_copy(gamma_hbm, gam_v)
        pltpu.sync_copy(beta_hbm, bet_v)
        glo = gam_v[pl.ds(0, LANES)]
        ghi = gam_v[pl.ds(LANES, LANES)]
        blo = bet_v[pl.ds(0, LANES)] * float(TOK)
        bhi = bet_v[pl.ds(LANES, LANES)] * float(TOK)
        idx15 = jnp.full((LANES,), LANES - 1, jnp.int32)

        idxs = (idx0, idx1)
        rows = (rows0, rows1)
        outs = (out0, out1)
        osems = (osem0, osem1)

        def split(c):
            return c // 2, base_b + (c % 2) * GB

        def idx_copies(c, b):
            f, b0 = split(c)
            return [
                pltpu.make_async_copy(
                    ids_hbm.at[f, l, pl.ds(b0, GB)],
                    idxs[b].at[l], isem)
                for l in range(TOK)
            ]

        def row_copies(c, b):
            return [
                pltpu.make_async_copy(
                    table_hbm.at[idxs[b].at[l]],
                    rows[b].at[pl.ds(l * GB, GB)],
                    rsem,
                )
                for l in range(TOK)
            ]

        def out_copy(c, b):
            f, b0 = split(c)
            return pltpu.make_async_copy(
                outs[b],
                out_hbm.at[pl.ds(b0, GB), pl.ds(f, 1)],
                osems[b])

        def lane_total(v):
            return _lane_perm(jnp.cumsum(v), idx15)

        def compute(b):
            rv = rows[b]
            ov = outs[b]

            def group(g, gcarry):
                acc_lo = jnp.zeros((LANES,), jnp.float32)
                acc_hi = jnp.zeros((LANES,), jnp.float32)
                for l in range(TOK):
                    r = l * GB + g
                    lo = rv[r, pl.ds(0, LANES)]
                    hi = rv[r, pl.ds(LANES, LANES)]
                    tot = lane_total(lo + hi)
                    tot2 = lane_total(lo * lo + hi * hi)
                    mean = tot * (1.0 / DIM)
                    mh = tot * (0.5 / DIM)
                    xh = tot2 * (0.5 / DIM) - mean * mh
                    inv = _newton_rsqrt_half(xh)
                    acc_lo = acc_lo + (lo - mean) * inv
                    acc_hi = acc_hi + (hi - mean) * inv
                ov[g, 0, pl.ds(0, LANES)] = acc_lo * glo + blo
                ov[g, 0, pl.ds(LANES, LANES)] = acc_hi * ghi + bhi
                return gcarry

            lax.fori_loop(0, GB, group, 0, unroll=2)

        def pair(c2, carry):
            for b in (0, 1):
                c = c2 * 2 + b
                for cp in row_copies(c, b):
                    cp.wait()

                @pl.when(c + 1 < NCHUNK)
                def _():
                    for cp in idx_copies(c + 1, b ^ 1):
                        cp.wait()
                    for cp in row_copies(c + 1, b ^ 1):
                        cp.start()

                @pl.when(c >= 2)
                def _():
                    out_copy(c - 2, b).wait()

                compute(b)

                @pl.when(c + 2 < NCHUNK)
                def _():
                    for cp in idx_copies(c + 2, b):
                        cp.start()

                out_copy(c, b).start()
            return carry

        for cp in idx_copies(0, 0):
            cp.start()
        for cp in idx_copies(0, 0):
            cp.wait()
        for cp in row_copies(0, 0):
            cp.start()
        for cp in idx_copies(1, 1):
            cp.start()
        lax.fori_loop(0, NCHUNK // 2, pair, 0)
        out_copy(NCHUNK - 2, 0).wait()
        out_copy(NCHUNK - 1, 1).wait()

    return sc_kernel


_SC_KERNEL = _make_sc_kernel()


def kernel(input_ids, table, gamma, beta):
    ids_t = input_ids.transpose(1, 2, 0)
    return _SC_KERNEL(ids_t, table, gamma, beta)


# speedup vs baseline: 1.0589x; 1.0139x over previous
import functools

import jax
import jax.numpy as jnp
from jax import lax
from jax.experimental import pallas as pl
from jax.experimental.pallas import tpu as pltpu
from jax.experimental.pallas import tpu_sc as plsc

B, F, TOK = 4096, 26, 20
DIM = 32
LANES = 16
NC, NS = 2, 16
NW = NC * NS
B_PW = B // NW
GB = 64
C = GB * TOK
NCHUNK = F * (B_PW // GB)
EPS = 1e-12
RSQRT_MAGIC = 0x5F3759DF

_GDN = lax.GatherDimensionNumbers(
    offset_dims=(), collapsed_slice_dims=(0,), start_index_map=(0,))


def _lane_perm(v, idx):
    return lax.gather(v, idx.reshape(LANES, 1), _GDN, (1,),
                      mode=lax.GatherScatterMode.PROMISE_IN_BOUNDS)


def _newton_rsqrt_half(xh):
    i = plsc.bitcast(xh, jnp.int32)
    i = (RSQRT_MAGIC - 0x400000) - lax.shift_right_logical(i, 1)
    y = plsc.bitcast(i, jnp.float32)
    y = y * (1.5 - xh * y * y)
    return y


def _make_sc_kernel():
    mesh = plsc.VectorSubcoreMesh(core_axis_name="c", subcore_axis_name="s")

    @functools.partial(
        pl.kernel,
        out_type=jax.ShapeDtypeStruct((B, F, DIM), jnp.float32),
        mesh=mesh,
        compiler_params=pltpu.CompilerParams(
            needs_layout_passes=False, use_tc_tiling_on_sc=False),
        scratch_types=[
            pltpu.VMEM((TOK, GB), jnp.int32),
            pltpu.VMEM((TOK, GB), jnp.int32),
            pltpu.VMEM((C, DIM), jnp.float32),
            pltpu.VMEM((C, DIM), jnp.float32),
            pltpu.VMEM((GB, 1, DIM), jnp.float32),
            pltpu.VMEM((GB, 1, DIM), jnp.float32),
            pltpu.VMEM((DIM,), jnp.float32),
            pltpu.VMEM((DIM,), jnp.float32),
            pltpu.SemaphoreType.DMA,
            pltpu.SemaphoreType.DMA,
            pltpu.SemaphoreType.DMA,
            pltpu.SemaphoreType.DMA,
        ],
    )
    def sc_kernel(ids_hbm, table_hbm, gamma_hbm, beta_hbm, out_hbm,
                  idx0, idx1, rows0, rows1, out0, out1, gam_v, bet_v,
                  isem, rsem, osem0, osem1):
        wid = lax.axis_index("s") * NC + lax.axis_index("c")
        base_b = wid * B_PW
        pltpu.sync_copy(gamma_hbm, gam_v)
        pltpu.sync_copy(beta_hbm, bet_v)
        glo = gam_v[pl.ds(0, LANES)]
        ghi = gam_v[pl.ds(LANES, LANES)]
        blo = bet_v[pl.ds(0, LANES)] * float(TOK)
        bhi = bet_v[pl.ds(LANES, LANES)] * float(TOK)
        idx15 = jnp.full((LANES,), LANES - 1, jnp.int32)

        idxs = (idx0, idx1)
        rows = (rows0, rows1)
        outs = (out0, out1)
        osems = (osem0, osem1)

        def split(c):
            return c // 2, base_b + (c % 2) * GB

        def idx_copies(c, b):
            f, b0 = split(c)
            return [
                pltpu.make_async_copy(
                    ids_hbm.at[f, l, pl.ds(b0, GB)],
                    idxs[b].at[l], isem)
                for l in range(TOK)
            ]

        def row_copies(c, b):
            return [
                pltpu.make_async_copy(
                    table_hbm.at[idxs[b].at[l]],
                    rows[b].at[pl.ds(l * GB, GB)],
                    rsem,
                )
                for l in range(TOK)
            ]

        def out_copy(c, b):
            f, b0 = split(c)
            return pltpu.make_async_copy(
                outs[b],
                out_hbm.at[pl.ds(b0, GB), pl.ds(f, 1)],
                osems[b])

        def lane_total(v):
            return _lane_perm(jnp.cumsum(v), idx15)

        def compute(b):
            rv = rows[b]
            ov = outs[b]

            def group(g, gcarry):
                acc_lo = jnp.zeros((LANES,), jnp.float32)
                acc_hi = jnp.zeros((LANES,), jnp.float32)
                for l in range(TOK):
                    r = l * GB + g
                    lo = rv[r, pl.ds(0, LANES)]
                    hi = rv[r, pl.ds(LANES, LANES)]
                    tot = lane_total(lo + hi)
                    tot2 = lane_total(lo * lo + hi * hi)
                    mean = tot * (1.0 / DIM)
                    mh = tot * (0.5 / DIM)
                    xh = tot2 * (0.5 / DIM) - mean * mh
                    inv = _newton_rsqrt_half(xh)
                    acc_lo = acc_lo + (lo - mean) * inv
                    acc_hi = acc_hi + (hi - mean) * inv
                ov[g, 0, pl.ds(0, LANES)] = acc_lo * glo + blo
                ov[g, 0, pl.ds(LANES, LANES)] = acc_hi * ghi + bhi
                return gcarry

            lax.fori_loop(0, GB, group, 0, unroll=4)

        def pair(c2, carry):
            for b in (0, 1):
                c = c2 * 2 + b
                for cp in row_copies(c, b):
                    cp.wait()

                @pl.when(c + 1 < NCHUNK)
                def _():
                    for cp in idx_copies(c + 1, b ^ 1):
                        cp.wait()
                    for cp in row_copies(c + 1, b ^ 1):
                        cp.start()

                @pl.when(c >= 2)
                def _():
                    out_copy(c - 2, b).wait()

                compute(b)

                @pl.when(c + 2 < NCHUNK)
                def _():
                    for cp in idx_copies(c + 2, b):
                        cp.start()

                out_copy(c, b).start()
            return carry

        for cp in idx_copies(0, 0):
            cp.start()
        for cp in idx_copies(0, 0):
            cp.wait()
        for cp in row_copies(0, 0):
            cp.start()
        for cp in idx_copies(1, 1):
            cp.start()
        lax.fori_loop(0, NCHUNK // 2, pair, 0)
        out_copy(NCHUNK - 2, 0).wait()
        out_copy(NCHUNK - 1, 1).wait()

    return sc_kernel


_SC_KERNEL = _make_sc_kernel()


def kernel(input_ids, table, gamma, beta):
    ids_t = input_ids.transpose(1, 2, 0)
    return _SC_KERNEL(ids_t, table, gamma, beta)
